# Initial kernel scaffold; baseline (speedup 1.0000x reference)
import jax
import jax.numpy as jnp
from jax.experimental import pallas as pl


def kernel(w, incoming_emb, mask, i, W1, b1, W2, b2):
    raise NotImplementedError("write your pallas kernel here")


# speedup vs baseline: 1.6424x; 1.6424x over previous
import functools

import jax
import jax.numpy as jnp
import numpy as np
from jax.experimental import pallas as pl
from jax.experimental.pallas import tpu as pltpu

VSIZE = 100.0
NEG = np.float32(-1e6)


def _step_kernel(b2_ref, embT_ref, wT_ref, maskT_ref, w1T_ref, b1_ref,
                 w2row_ref, pT_out, sel_out, logp_out, s_ref):
    d2 = s_ref.shape[0]
    s_ref[1:d2, :] = embT_ref[...]
    u1 = wT_ref.shape[0]
    b2 = b2_ref[0]
    b1c = b1_ref[...]
    rows = []
    for u in range(u1):
        s_ref[0:1, :] = wT_ref[u:u + 1, :]
        h = jnp.dot(w1T_ref[...], s_ref[...],
                    preferred_element_type=jnp.float32) + b1c
        act = jnp.maximum(h, 0.0).astype(jnp.bfloat16)
        rows.append(jnp.dot(w2row_ref[...], act,
                            preferred_element_type=jnp.float32) + b2)
    pi = jnp.concatenate(rows, axis=0)
    maskv = maskT_ref[...]
    pi = jnp.where(maskv > 0.5, NEG, pi)
    m1 = jnp.max(pi, axis=0, keepdims=True)
    lse = jnp.log(jnp.sum(jnp.exp(pi - m1), axis=0, keepdims=True))
    p = pi - m1 - lse
    pT_out[...] = p
    iota = jax.lax.broadcasted_iota(jnp.int32, pi.shape, 0)
    big = jnp.int32(2 * u1)
    sel_out[...] = jnp.min(jnp.where(p == jnp.max(p, axis=0, keepdims=True),
                                     iota, big), axis=0, keepdims=True)
    logp_out[...] = jnp.max(p, axis=0, keepdims=True)


@functools.partial(jax.jit, static_argnames=())
def kernel(w, incoming_emb, mask, i, W1, b1, W2, b2):
    B, U1 = w.shape
    D = incoming_emb.shape[-1]
    H = W1.shape[-1]
    H2 = ((H + 7) // 8) * 8
    D2 = D + 2

    def r16(t):
        return t.astype(jnp.bfloat16).astype(jnp.float32)

    b1col = jnp.concatenate([b1, jnp.zeros((H2 - H,), jnp.float32)])[:, None]
    w1T = jnp.concatenate([r16(W1.T),
                           jnp.zeros((H2 - H, D2), jnp.float32)], axis=0)
    w2row = jnp.concatenate([r16(W2[:, 0]),
                             jnp.zeros((H2 - H,), jnp.float32)])[None, :]

    embT = jnp.concatenate(
        [jnp.broadcast_to(r16(jnp.float32(i) / VSIZE), (1, B)),
         r16(incoming_emb.T)], axis=0)
    wT = r16(w.T)
    maskT = mask.T.astype(jnp.float32)

    bB = 512
    grid = (B // bB,)
    pT, sel, logp = pl.pallas_call(
        _step_kernel,
        grid=grid,
        in_specs=[
            pl.BlockSpec(memory_space=pltpu.SMEM),
            pl.BlockSpec((1 + D, bB), lambda j: (0, j)),
            pl.BlockSpec((U1, bB), lambda j: (0, j)),
            pl.BlockSpec((U1, bB), lambda j: (0, j)),
            pl.BlockSpec((H2, D2), lambda j: (0, 0)),
            pl.BlockSpec((H2, 1), lambda j: (0, 0)),
            pl.BlockSpec((1, H2), lambda j: (0, 0)),
        ],
        out_specs=[
            pl.BlockSpec((U1, bB), lambda j: (0, j)),
            pl.BlockSpec((1, bB), lambda j: (0, j)),
            pl.BlockSpec((1, bB), lambda j: (0, j)),
        ],
        out_shape=[
            jax.ShapeDtypeStruct((U1, B), jnp.float32),
            jax.ShapeDtypeStruct((1, B), jnp.int32),
            jax.ShapeDtypeStruct((1, B), jnp.float32),
        ],
        scratch_shapes=[pltpu.VMEM((D2, bB), jnp.float32)],
    )(b2, embT, wT, maskT, w1T, b1col, w2row)
    return pT.T, sel[0], logp[0]


# speedup vs baseline: 2.5036x; 1.5244x over previous
import functools

import jax
import jax.numpy as jnp
import numpy as np
from jax.experimental import pallas as pl
from jax.experimental.pallas import tpu as pltpu

VSIZE = 100.0
NEG = np.float32(-1e6)


def _step_kernel(b2_ref, embT_ref, wT_ref, maskT_ref, w1T_ref, b1_ref,
                 w2row_ref, pT_out, sel_out, logp_out, s_ref):
    d2 = s_ref.shape[0]
    s_ref[1:d2, :] = embT_ref[...]
    u1 = wT_ref.shape[0]
    b2 = b2_ref[0]
    b1c = b1_ref[...]
    rows = []
    for u in range(u1):
        s_ref[0:1, :] = wT_ref[u:u + 1, :]
        h = jnp.dot(w1T_ref[...], s_ref[...],
                    preferred_element_type=jnp.float32) + b1c
        act = jnp.maximum(h, 0.0).astype(jnp.bfloat16)
        rows.append(jnp.dot(w2row_ref[...], act,
                            preferred_element_type=jnp.float32) + b2)
    pi = jnp.concatenate(rows, axis=0)
    maskv = maskT_ref[...]
    pi = jnp.where(maskv > 0.5, NEG, pi)
    m1 = jnp.max(pi, axis=0, keepdims=True)
    lse = jnp.log(jnp.sum(jnp.exp(pi - m1), axis=0, keepdims=True))
    p = pi - m1 - lse
    pT_out[...] = p
    iota = jax.lax.broadcasted_iota(jnp.int32, pi.shape, 0)
    big = jnp.int32(2 * u1)
    sel_out[...] = jnp.min(jnp.where(p == jnp.max(p, axis=0, keepdims=True),
                                     iota, big), axis=0, keepdims=True)
    logp_out[...] = jnp.max(p, axis=0, keepdims=True)


@functools.partial(jax.jit, static_argnames=())
def kernel(w, incoming_emb, mask, i, W1, b1, W2, b2):
    B, U1 = w.shape
    D = incoming_emb.shape[-1]
    H = W1.shape[-1]
    H2 = ((H + 7) // 8) * 8
    D2 = D + 2

    def r16(t):
        return t.astype(jnp.bfloat16).astype(jnp.float32)

    b1col = jnp.concatenate([b1, jnp.zeros((H2 - H,), jnp.float32)])[:, None]
    w1T = jnp.concatenate([r16(W1.T),
                           jnp.zeros((H2 - H, D2), jnp.float32)], axis=0)
    w2row = jnp.concatenate([r16(W2[:, 0]),
                             jnp.zeros((H2 - H,), jnp.float32)])[None, :]

    embT = jnp.concatenate(
        [jnp.broadcast_to(r16(jnp.float32(i) / VSIZE), (1, B)),
         r16(incoming_emb.T)], axis=0)
    wT = r16(w.T)
    maskT = mask.T.astype(jnp.float32)

    bB = 1024
    grid = (B // bB,)
    pT, sel, logp = pl.pallas_call(
        _step_kernel,
        grid=grid,
        in_specs=[
            pl.BlockSpec(memory_space=pltpu.SMEM),
            pl.BlockSpec((1 + D, bB), lambda j: (0, j)),
            pl.BlockSpec((U1, bB), lambda j: (0, j)),
            pl.BlockSpec((U1, bB), lambda j: (0, j)),
            pl.BlockSpec((H2, D2), lambda j: (0, 0)),
            pl.BlockSpec((H2, 1), lambda j: (0, 0)),
            pl.BlockSpec((1, H2), lambda j: (0, 0)),
        ],
        out_specs=[
            pl.BlockSpec((U1, bB), lambda j: (0, j)),
            pl.BlockSpec((1, bB), lambda j: (0, j)),
            pl.BlockSpec((1, bB), lambda j: (0, j)),
        ],
        out_shape=[
            jax.ShapeDtypeStruct((U1, B), jnp.float32),
            jax.ShapeDtypeStruct((1, B), jnp.int32),
            jax.ShapeDtypeStruct((1, B), jnp.float32),
        ],
        scratch_shapes=[pltpu.VMEM((D2, bB), jnp.float32)],
    )(b2, embT, wT, maskT, w1T, b1col, w2row)
    return pT.T, sel[0], logp[0]


# speedup vs baseline: 2.5855x; 1.0327x over previous
import functools

import jax
import jax.numpy as jnp
import numpy as np
from jax.experimental import pallas as pl
from jax.experimental.pallas import tpu as pltpu

VSIZE = 100.0
NEG = np.float32(-1e6)


def _step_kernel(b2_ref, embT_ref, wT_ref, maskT_ref, w1T_ref, b1_ref,
                 w2row_ref, pT_out, sel_out, logp_out, s_ref):
    d2 = s_ref.shape[0]
    s_ref[1:d2, :] = embT_ref[...]
    u1 = wT_ref.shape[0]
    b2 = b2_ref[0]
    b1c = b1_ref[...]
    rows = []
    for u in range(u1):
        s_ref[0:1, :] = wT_ref[u:u + 1, :]
        h = jnp.dot(w1T_ref[...], s_ref[...],
                    preferred_element_type=jnp.float32) + b1c
        act = jnp.maximum(h, 0.0).astype(jnp.bfloat16)
        rows.append(jnp.dot(w2row_ref[...], act,
                            preferred_element_type=jnp.float32) + b2)
    pi = jnp.concatenate(rows, axis=0)
    maskv = maskT_ref[...]
    pi = jnp.where(maskv > 0.5, NEG, pi)
    m1 = jnp.max(pi, axis=0, keepdims=True)
    lse = jnp.log(jnp.sum(jnp.exp(pi - m1), axis=0, keepdims=True))
    p = pi - m1 - lse
    pT_out[...] = p
    iota = jax.lax.broadcasted_iota(jnp.int32, pi.shape, 0)
    big = jnp.int32(2 * u1)
    sel_out[...] = jnp.min(jnp.where(p == jnp.max(p, axis=0, keepdims=True),
                                     iota, big), axis=0, keepdims=True)
    logp_out[...] = jnp.max(p, axis=0, keepdims=True)


@functools.partial(jax.jit, static_argnames=())
def kernel(w, incoming_emb, mask, i, W1, b1, W2, b2):
    B, U1 = w.shape
    D = incoming_emb.shape[-1]
    H = W1.shape[-1]
    H2 = ((H + 7) // 8) * 8
    D2 = D + 2

    def r16(t):
        return t.astype(jnp.bfloat16).astype(jnp.float32)

    b1col = jnp.concatenate([b1, jnp.zeros((H2 - H,), jnp.float32)])[:, None]
    w1T = jnp.concatenate([r16(W1.T),
                           jnp.zeros((H2 - H, D2), jnp.float32)], axis=0)
    w2row = jnp.concatenate([r16(W2[:, 0]),
                             jnp.zeros((H2 - H,), jnp.float32)])[None, :]

    embT = jnp.concatenate(
        [jnp.broadcast_to(r16(jnp.float32(i) / VSIZE), (1, B)),
         r16(incoming_emb.T)], axis=0)
    wT = r16(w.T)
    maskT = mask.T.astype(jnp.float32)

    bB = 2048
    grid = (B // bB,)
    pT, sel, logp = pl.pallas_call(
        _step_kernel,
        grid=grid,
        in_specs=[
            pl.BlockSpec(memory_space=pltpu.SMEM),
            pl.BlockSpec((1 + D, bB), lambda j: (0, j)),
            pl.BlockSpec((U1, bB), lambda j: (0, j)),
            pl.BlockSpec((U1, bB), lambda j: (0, j)),
            pl.BlockSpec((H2, D2), lambda j: (0, 0)),
            pl.BlockSpec((H2, 1), lambda j: (0, 0)),
            pl.BlockSpec((1, H2), lambda j: (0, 0)),
        ],
        out_specs=[
            pl.BlockSpec((U1, bB), lambda j: (0, j)),
            pl.BlockSpec((1, bB), lambda j: (0, j)),
            pl.BlockSpec((1, bB), lambda j: (0, j)),
        ],
        out_shape=[
            jax.ShapeDtypeStruct((U1, B), jnp.float32),
            jax.ShapeDtypeStruct((1, B), jnp.int32),
            jax.ShapeDtypeStruct((1, B), jnp.float32),
        ],
        scratch_shapes=[pltpu.VMEM((D2, bB), jnp.float32)],
    )(b2, embT, wT, maskT, w1T, b1col, w2row)
    return pT.T, sel[0], logp[0]
